# baseline (speedup 1.0000x reference)
import functools

import jax
import jax.numpy as jnp
from jax import lax
from jax.experimental import pallas as pl
from jax.experimental.pallas import tpu as pltpu
from jax.experimental.pallas import tpu_sc as plsc

B, T, D = 1, 2048, 768
E, K, FF = 8, 2, 1024
BT = B * T
TILE = 256
N_TILES = 24
N_BUF = N_TILES * TILE
NW = 32
ROWS_W = K * BT // NW


def _router_body(x_ref, gate_ref, xc_ref, slots_ref, w1c_ref, w2c_ref,
                 wtc_ref, base_ref, aux_ref):
    x = x_ref[...]
    mu = jnp.mean(x, axis=1, keepdims=True)
    var = jnp.mean((x - mu) ** 2, axis=1, keepdims=True)
    xc_ref[...] = (x - mu) * lax.rsqrt(var + 1e-6)

    logits = lax.dot_general(x, gate_ref[...], (((1,), (1,)), ((), ())),
                             preferred_element_type=jnp.float32)
    m = jnp.max(logits, axis=1, keepdims=True)
    ex = jnp.exp(logits - m)
    s = ex / jnp.sum(ex, axis=1, keepdims=True)
    eids = lax.broadcasted_iota(jnp.int32, (BT, E), 1)
    v1 = jnp.max(s, axis=1, keepdims=True)
    i1 = jnp.min(jnp.where(s == v1, eids, E), axis=1, keepdims=True)
    s_m = jnp.where(eids == i1, -1.0, s)
    v2 = jnp.max(s_m, axis=1, keepdims=True)
    i2 = jnp.min(jnp.where(s_m == v2, eids, E), axis=1, keepdims=True)
    sel = (eids == i1) | (eids == i2)
    self32 = sel.astype(jnp.float32)

    w1c_ref[...] = v1
    w2c_ref[...] = v2
    wtc_ref[...] = v1 + v2

    load = jnp.sum(self32, axis=0, keepdims=True)
    importance = jnp.sum(s, axis=0, keepdims=True)
    aux = jnp.sum(load * importance) * (E / (BT * BT))
    aux_ref[...] = jnp.broadcast_to(aux, (1, 1))

    cnt = load
    pc = jnp.floor((cnt + (TILE - 1)) / TILE) * TILE
    tri8 = (lax.broadcasted_iota(jnp.int32, (E, E), 0)
            < lax.broadcasted_iota(jnp.int32, (E, E), 1)).astype(jnp.float32)
    base = lax.dot_general(pc, tri8, (((1,), (0,)), ((), ())),
                           preferred_element_type=jnp.float32)
    base_ref[...] = base.astype(jnp.int32)

    ltri = (lax.broadcasted_iota(jnp.int32, (BT, BT), 1)
            < lax.broadcasted_iota(jnp.int32, (BT, BT), 0)).astype(jnp.bfloat16)
    p_excl = lax.dot_general(ltri, self32.astype(jnp.bfloat16),
                             (((1,), (0,)), ((), ())),
                             preferred_element_type=jnp.float32)
    slotf = p_excl + base
    slot1 = jnp.sum(jnp.where(eids == i1, slotf, 0.0), axis=1, keepdims=True)
    slot2 = jnp.sum(jnp.where(eids == i2, slotf, 0.0), axis=1, keepdims=True)
    slots_ref[...] = jnp.concatenate(
        [slot1.astype(jnp.int32), slot2.astype(jnp.int32)], axis=1)


def _router(xf, gate_w):
    return pl.pallas_call(
        _router_body,
        out_specs=[
            pl.BlockSpec((BT, D), lambda: (0, 0)),
            pl.BlockSpec((BT, K), lambda: (0, 0)),
            pl.BlockSpec((BT, 1), lambda: (0, 0)),
            pl.BlockSpec((BT, 1), lambda: (0, 0)),
            pl.BlockSpec((BT, 1), lambda: (0, 0)),
            pl.BlockSpec((1, E), lambda: (0, 0)),
            pl.BlockSpec((1, 1), lambda: (0, 0)),
        ],
        out_shape=[
            jax.ShapeDtypeStruct((BT, D), jnp.float32),
            jax.ShapeDtypeStruct((BT, K), jnp.int32),
            jax.ShapeDtypeStruct((BT, 1), jnp.float32),
            jax.ShapeDtypeStruct((BT, 1), jnp.float32),
            jax.ShapeDtypeStruct((BT, 1), jnp.float32),
            jax.ShapeDtypeStruct((1, E), jnp.int32),
            jax.ShapeDtypeStruct((1, 1), jnp.float32),
        ],
    )(xf, gate_w)


@functools.cache
def _sc_kernels():
    mesh = plsc.VectorSubcoreMesh(core_axis_name="c", subcore_axis_name="s")

    @functools.partial(
        pl.kernel, mesh=mesh,
        out_type=jax.ShapeDtypeStruct((N_BUF, D), jnp.float32),
        scratch_types=[
            pltpu.VMEM((1, ROWS_W), jnp.int32),
            pltpu.VMEM((ROWS_W, D), jnp.float32),
            pltpu.SemaphoreType.DMA,
        ],
    )
    def _sc_scatter(xc_hbm, slot3d_hbm, xs_hbm, idx_v, rows_v, sem):
        wid = lax.axis_index("s") * 2 + lax.axis_index("c")
        t0 = (wid % 16) * ROWS_W
        pltpu.sync_copy(slot3d_hbm.at[wid], idx_v)
        pltpu.sync_copy(xc_hbm.at[pl.ds(t0, ROWS_W)], rows_v)
        pltpu.async_copy(rows_v, xs_hbm.at[idx_v.at[0]], sem).wait()

    @functools.partial(
        pl.kernel, mesh=mesh,
        out_type=jax.ShapeDtypeStruct((K * BT, D), jnp.float32),
        scratch_types=[
            pltpu.VMEM((1, ROWS_W), jnp.int32),
            pltpu.VMEM((ROWS_W, D), jnp.float32),
            pltpu.SemaphoreType.DMA,
        ],
    )
    def _sc_gather(ys_hbm, slot3d_hbm, g_hbm, idx_v, rows_v, sem):
        wid = lax.axis_index("s") * 2 + lax.axis_index("c")
        pltpu.sync_copy(slot3d_hbm.at[wid], idx_v)
        pltpu.async_copy(ys_hbm.at[idx_v.at[0]], rows_v, sem).wait()
        pltpu.sync_copy(rows_v, g_hbm.at[pl.ds(wid * ROWS_W, ROWS_W)])

    return _sc_scatter, _sc_gather


def _ffn_body(te_ref, xs_ref, lng_ref, lnb_ref, w1_ref, b1_ref, w2_ref,
              b2_ref, ys_ref):
    xn = xs_ref[...] * lng_ref[0] + lnb_ref[0]
    h = lax.dot_general(xn, w1_ref[0], (((1,), (1,)), ((), ())),
                        preferred_element_type=jnp.float32)
    h = jnp.maximum(h + b1_ref[0], 0.0)
    y = lax.dot_general(h, w2_ref[0], (((1,), (1,)), ((), ())),
                        preferred_element_type=jnp.float32)
    ys_ref[...] = y + b2_ref[0]


def _ffn(te, xs, ln_g, ln_b, W1, b1, W2, b2):
    grid_spec = pltpu.PrefetchScalarGridSpec(
        num_scalar_prefetch=1,
        grid=(N_TILES,),
        in_specs=[
            pl.BlockSpec((TILE, D), lambda j, te: (j, 0)),
            pl.BlockSpec((1, 1, D), lambda j, te: (te[j], 0, 0)),
            pl.BlockSpec((1, 1, D), lambda j, te: (te[j], 0, 0)),
            pl.BlockSpec((1, FF, D), lambda j, te: (te[j], 0, 0)),
            pl.BlockSpec((1, 1, FF), lambda j, te: (te[j], 0, 0)),
            pl.BlockSpec((1, D, FF), lambda j, te: (te[j], 0, 0)),
            pl.BlockSpec((1, 1, D), lambda j, te: (te[j], 0, 0)),
        ],
        out_specs=pl.BlockSpec((TILE, D), lambda j, te: (j, 0)),
    )
    return pl.pallas_call(
        _ffn_body,
        grid_spec=grid_spec,
        out_shape=jax.ShapeDtypeStruct((N_BUF, D), jnp.float32),
        compiler_params=pltpu.CompilerParams(
            dimension_semantics=("arbitrary",),
        ),
    )(te, xs, ln_g.reshape(E, 1, D), ln_b.reshape(E, 1, D), W1,
      b1.reshape(E, 1, FF), W2, b2.reshape(E, 1, D))


def _combine_body(g_ref, w1_ref, w2_ref, wt_ref, xf_ref, out_ref):
    g1 = g_ref[0:BT]
    g2 = g_ref[BT:2 * BT]
    out_ref[...] = (w1_ref[...] * g1 + w2_ref[...] * g2
                    + wt_ref[...] * xf_ref[...])


def _combine(g, w1c, w2c, wtc, xf):
    return pl.pallas_call(
        _combine_body,
        out_shape=jax.ShapeDtypeStruct((BT, D), jnp.float32),
    )(g, w1c, w2c, wtc, xf)


@jax.jit
def kernel(x, gate_w, ln_g, ln_b, W1, b1, W2, b2):
    xf = x.reshape(BT, D)
    xc, slots, w1c, w2c, wtc, base_i, aux = _router(xf, gate_w)
    slot3d = slots.T.reshape(NW, 1, ROWS_W)
    tile_start = jnp.arange(N_TILES, dtype=jnp.int32)[:, None] * TILE
    te = jnp.sum((tile_start >= base_i[0][None, :]).astype(jnp.int32),
                 axis=1) - 1
    sc_scatter, sc_gather = _sc_kernels()
    xs = sc_scatter(xc, slot3d)
    ys = _ffn(te, xs, ln_g, ln_b, W1, b1, W2, b2)
    g = sc_gather(ys, slot3d)
    out = _combine(g, w1c, w2c, wtc, xf)
    return out.reshape(B, T, D), aux[0, 0]


# speedup vs baseline: 1.4405x; 1.4405x over previous
import functools

import jax
import jax.numpy as jnp
from jax.experimental import pallas as pl
from jax.experimental.pallas import tpu as pltpu

B, T, D = 1, 2048, 768
E, K, FF = 8, 2, 1024
BT = B * T


def _moe_body(x_ref, gate_ref, ln_g_ref, ln_b_ref, w1_ref, b1_ref,
              w2_ref, b2_ref, out_ref, aux_ref, xc_s, wmask_s, wtot_s):
    e = pl.program_id(0)

    @pl.when(e == 0)
    def _router():
        x = x_ref[...]
        mu = jnp.mean(x, axis=1, keepdims=True)
        var = jnp.mean((x - mu) ** 2, axis=1, keepdims=True)
        xc_s[...] = (x - mu) * jax.lax.rsqrt(var + 1e-6)
        logits = jax.lax.dot_general(
            x, gate_ref[...], (((1,), (1,)), ((), ())),
            preferred_element_type=jnp.float32)
        m = jnp.max(logits, axis=1, keepdims=True)
        ex = jnp.exp(logits - m)
        s = ex / jnp.sum(ex, axis=1, keepdims=True)
        eids = jax.lax.broadcasted_iota(jnp.int32, (BT, E), 1)
        v1 = jnp.max(s, axis=1, keepdims=True)
        i1 = jnp.min(jnp.where(s == v1, eids, E), axis=1, keepdims=True)
        s_m = jnp.where(eids == i1, -1.0, s)
        v2 = jnp.max(s_m, axis=1, keepdims=True)
        i2 = jnp.min(jnp.where(s_m == v2, eids, E), axis=1, keepdims=True)
        sel = (eids == i1) | (eids == i2)
        wmask = jnp.where(sel, s, 0.0)
        wmask_s[...] = wmask
        wtot_s[...] = jnp.sum(wmask, axis=1, keepdims=True)
        load = jnp.sum(sel.astype(jnp.float32), axis=0)
        importance = jnp.sum(s, axis=0)
        aux = jnp.sum(load * importance) * (E / (BT * BT))
        aux_ref[...] = jnp.broadcast_to(aux, (1, 1))

    xn = xc_s[...] * ln_g_ref[0] + ln_b_ref[0]
    h = jax.lax.dot_general(
        xn.astype(jnp.bfloat16), w1_ref[0].astype(jnp.bfloat16),
        (((1,), (1,)), ((), ())),
        preferred_element_type=jnp.float32)
    h = jnp.maximum(h + b1_ref[0], 0.0)
    y = jax.lax.dot_general(
        h.astype(jnp.bfloat16), w2_ref[0].astype(jnp.bfloat16),
        (((1,), (1,)), ((), ())),
        preferred_element_type=jnp.float32)
    y = y + b2_ref[0]
    lane = jax.lax.broadcasted_iota(jnp.int32, (BT, E), 1)
    w_col = jnp.sum(jnp.where(lane == e, wmask_s[...], 0.0),
                    axis=1, keepdims=True)

    @pl.when(e == 0)
    def _init():
        out_ref[...] = w_col * y + wtot_s[...] * x_ref[...]

    @pl.when(e != 0)
    def _acc():
        out_ref[...] = out_ref[...] + w_col * y


@jax.jit
def kernel(x, gate_w, ln_g, ln_b, W1, b1, W2, b2):
    xf = x.reshape(BT, D)
    out, aux = pl.pallas_call(
        _moe_body,
        grid=(E,),
        in_specs=[
            pl.BlockSpec((BT, D), lambda e: (0, 0)),
            pl.BlockSpec((E, D), lambda e: (0, 0)),
            pl.BlockSpec((1, 1, D), lambda e: (e, 0, 0)),
            pl.BlockSpec((1, 1, D), lambda e: (e, 0, 0)),
            pl.BlockSpec((1, FF, D), lambda e: (e, 0, 0)),
            pl.BlockSpec((1, 1, FF), lambda e: (e, 0, 0)),
            pl.BlockSpec((1, D, FF), lambda e: (e, 0, 0)),
            pl.BlockSpec((1, 1, D), lambda e: (e, 0, 0)),
        ],
        out_specs=[
            pl.BlockSpec((BT, D), lambda e: (0, 0)),
            pl.BlockSpec((1, 1), lambda e: (0, 0)),
        ],
        out_shape=[
            jax.ShapeDtypeStruct((BT, D), jnp.float32),
            jax.ShapeDtypeStruct((1, 1), jnp.float32),
        ],
        scratch_shapes=[
            pltpu.VMEM((BT, D), jnp.float32),
            pltpu.VMEM((BT, E), jnp.float32),
            pltpu.VMEM((BT, 1), jnp.float32),
        ],
        compiler_params=pltpu.CompilerParams(
            dimension_semantics=("arbitrary",),
        ),
    )(xf, gate_w, ln_g.reshape(E, 1, D), ln_b.reshape(E, 1, D), W1,
      b1.reshape(E, 1, FF), W2, b2.reshape(E, 1, D))
    return out.reshape(B, T, D), aux[0, 0]
